# Initial kernel scaffold; baseline (speedup 1.0000x reference)
import jax
import jax.numpy as jnp
from jax.experimental import pallas as pl


def kernel(x, edge_index, edge_weight, W1, b1, W2, b2):
    raise NotImplementedError("write your pallas kernel here")


# speedup vs baseline: 9.3021x; 9.3021x over previous
import functools

import jax
import jax.numpy as jnp
from jax import lax
from jax.experimental import pallas as pl
from jax.experimental.pallas import tpu as pltpu
from jax.experimental.pallas import tpu_sc as plsc

N = 10000
E = 320000
D_IN = 128
D_H = 256
D_OUT = 64

NPAD = 10240
NC = 2
NS = 16
NW = NC * NS
CHUNK = 128
CHUNKS = -(-E // (NW * CHUNK))
EPW = CHUNKS * CHUNK
EPAD = NW * EPW
CH16 = -(-E // (NS * CHUNK))
ESUB = CH16 * CHUNK
EPAD16 = NS * ESUB
ROWS_PER_SUB = NPAD // NS

_mesh = plsc.VectorSubcoreMesh(
    core_axis_name="c", subcore_axis_name="s", num_cores=NC, num_subcores=NS)
_sc_params = pltpu.CompilerParams(
    needs_layout_passes=False, use_tc_tiling_on_sc=False)


@functools.partial(
    pl.kernel,
    out_type=jax.ShapeDtypeStruct((NW, NPAD), jnp.float32),
    mesh=_mesh,
    compiler_params=_sc_params,
    scratch_types=[
        pltpu.VMEM((EPW,), jnp.int32),
        pltpu.VMEM((EPW,), jnp.float32),
        pltpu.VMEM((NPAD,), jnp.float32),
    ],
)
def _deg_kernel(dst_hbm, ew_hbm, out_hbm, dstb, ewb, degl):
    cid = lax.axis_index("c")
    sid = lax.axis_index("s")
    wid = sid * NC + cid
    pltpu.sync_copy(dst_hbm.at[wid], dstb)
    pltpu.sync_copy(ew_hbm.at[wid], ewb)

    def zero(i, _):
        degl[pl.ds(i * 16, 16)] = jnp.zeros((16,), jnp.float32)
        return 0

    lax.fori_loop(0, NPAD // 16, zero, 0)

    def body(g, _):
        idx = dstb[pl.ds(g * 16, 16)]
        w = ewb[pl.ds(g * 16, 16)]
        plsc.addupdate_scatter(degl, [idx], w)
        return 0

    lax.fori_loop(0, EPW // 16, body, 0)
    pltpu.sync_copy(degl, out_hbm.at[wid])


def _make_agg_kernel(D2):
    @functools.partial(
        pl.kernel,
        out_type=jax.ShapeDtypeStruct((NC, NPAD, D2), jnp.float32),
        mesh=_mesh,
        compiler_params=_sc_params,
        scratch_types=[
            pltpu.VMEM((CH16, CHUNK), jnp.int32),
            pltpu.VMEM((CH16, CHUNK), jnp.int32),
            pltpu.VMEM((CHUNK, 16), jnp.float32),
            pltpu.VMEM((CHUNK, D2), jnp.float32),
            pltpu.VMEM_SHARED((NPAD, D2), jnp.float32),
            pltpu.SemaphoreType.DMA,
        ],
    )
    def agg(table_hbm, src_hbm, dst_hbm, w_hbm, zeros_hbm, out_hbm,
            srcb, dstb, wbuf, rows, acc, sem):
        cid = lax.axis_index("c")
        sid = lax.axis_index("s")
        rbase = sid * ROWS_PER_SUB
        pltpu.sync_copy(zeros_hbm.at[pl.ds(rbase, ROWS_PER_SUB)],
                        acc.at[pl.ds(rbase, ROWS_PER_SUB)])
        pltpu.sync_copy(src_hbm.at[sid], srcb)
        pltpu.sync_copy(dst_hbm.at[sid], dstb)
        plsc.subcore_barrier()

        def chunk(ci, _):
            pltpu.async_copy(table_hbm.at[cid].at[srcb.at[ci]],
                             rows, sem).wait()
            pltpu.sync_copy(w_hbm.at[sid * CH16 + ci], wbuf)

            def scale(e, _):
                wv = wbuf[e, :]
                for j in range(D2 // 16):
                    sl = pl.ds(j * 16, 16)
                    rows[e, sl] = rows[e, sl] * wv
                return 0

            lax.fori_loop(0, CHUNK, scale, 0)
            pltpu.sync_copy(rows, acc.at[dstb.at[ci]], add=True)
            return 0

        lax.fori_loop(0, CH16, chunk, 0)
        plsc.subcore_barrier()
        pltpu.sync_copy(acc.at[pl.ds(rbase, ROWS_PER_SUB)],
                        out_hbm.at[cid, pl.ds(rbase, ROWS_PER_SUB)])

    return agg


_agg1 = _make_agg_kernel(D_IN // 2)
_agg2 = _make_agg_kernel(D_OUT // 2)


RB = 1024
NBB = NPAD // RB


H_IN = D_IN // 2
H_OUT = D_OUT // 2


def _prep_body(degp_ref, x_ref, dis_ref, xs_ref):
    deg = jnp.sum(degp_ref[...], axis=0, keepdims=True) + 1.0
    disr = lax.rsqrt(deg)
    r = lax.broadcasted_iota(jnp.int32, (RB, RB), 0)
    c = lax.broadcasted_iota(jnp.int32, (RB, RB), 1)
    dis_col = jnp.sum(
        jnp.where(r == c, jnp.broadcast_to(disr, (RB, RB)), 0.0),
        axis=1, keepdims=True)
    dis_ref[...] = dis_col
    xs = x_ref[...] * dis_col
    xs_ref[0] = xs[:, :H_IN]
    xs_ref[1] = xs[:, H_IN:]


def _mid_body(acc_ref, xs_ref, dis_ref, w1_ref, b1_ref, w2_ref, zs_ref):
    dis = dis_ref[...]
    t = jnp.concatenate(
        [acc_ref[0] + xs_ref[0], acc_ref[1] + xs_ref[1]], axis=1) * dis
    h = jnp.maximum(
        jnp.dot(t, w1_ref[...], preferred_element_type=jnp.float32)
        + b1_ref[...], 0.0)
    z = jnp.dot(h, w2_ref[...], preferred_element_type=jnp.float32)
    zs = z * dis
    zs_ref[0] = zs[:, :H_OUT]
    zs_ref[1] = zs[:, H_OUT:]


def _fin_body(acc_ref, zs_ref, dis_ref, b2_ref, out_ref):
    t = jnp.concatenate(
        [acc_ref[0] + zs_ref[0], acc_ref[1] + zs_ref[1]], axis=1)
    out_ref[...] = t * dis_ref[...] + b2_ref[...]


_prep_call = pl.pallas_call(
    _prep_body,
    grid=(NBB,),
    in_specs=[
        pl.BlockSpec((NW, RB), lambda r: (0, r)),
        pl.BlockSpec((RB, D_IN), lambda r: (r, 0)),
    ],
    out_specs=[
        pl.BlockSpec((RB, 1), lambda r: (r, 0)),
        pl.BlockSpec((NC, RB, H_IN), lambda r: (0, r, 0)),
    ],
    out_shape=[
        jax.ShapeDtypeStruct((NPAD, 1), jnp.float32),
        jax.ShapeDtypeStruct((NC, NPAD, H_IN), jnp.float32),
    ],
)

_mid_call = pl.pallas_call(
    _mid_body,
    grid=(NBB,),
    in_specs=[
        pl.BlockSpec((NC, RB, H_IN), lambda r: (0, r, 0)),
        pl.BlockSpec((NC, RB, H_IN), lambda r: (0, r, 0)),
        pl.BlockSpec((RB, 1), lambda r: (r, 0)),
        pl.BlockSpec((D_IN, D_H), lambda r: (0, 0)),
        pl.BlockSpec((1, D_H), lambda r: (0, 0)),
        pl.BlockSpec((D_H, D_OUT), lambda r: (0, 0)),
    ],
    out_specs=pl.BlockSpec((NC, RB, H_OUT), lambda r: (0, r, 0)),
    out_shape=jax.ShapeDtypeStruct((NC, NPAD, H_OUT), jnp.float32),
)

_fin_call = pl.pallas_call(
    _fin_body,
    grid=(NBB,),
    in_specs=[
        pl.BlockSpec((NC, RB, H_OUT), lambda r: (0, r, 0)),
        pl.BlockSpec((NC, RB, H_OUT), lambda r: (0, r, 0)),
        pl.BlockSpec((RB, 1), lambda r: (r, 0)),
        pl.BlockSpec((1, D_OUT), lambda r: (0, 0)),
    ],
    out_specs=pl.BlockSpec((RB, D_OUT), lambda r: (r, 0)),
    out_shape=jax.ShapeDtypeStruct((NPAD, D_OUT), jnp.float32),
)


def kernel(x, edge_index, edge_weight, W1, b1, W2, b2):
    src = edge_index[0].astype(jnp.int32)
    dst = edge_index[1].astype(jnp.int32)
    ew = edge_weight.astype(jnp.float32)

    pad = EPAD - E
    dst2 = jnp.concatenate(
        [dst, jnp.zeros((pad,), jnp.int32)]).reshape(NW, EPW)
    ew2 = jnp.concatenate(
        [ew, jnp.zeros((pad,), jnp.float32)]).reshape(NW, EPW)

    pad16 = EPAD16 - E
    src3 = jnp.concatenate(
        [src, jnp.zeros((pad16,), jnp.int32)]).reshape(NS, CH16, CHUNK)
    dst3 = jnp.concatenate(
        [dst, jnp.zeros((pad16,), jnp.int32)]).reshape(NS, CH16, CHUNK)
    ew16 = jnp.concatenate([ew, jnp.zeros((pad16,), jnp.float32)])
    wrep = jnp.broadcast_to(ew16[:, None], (EPAD16, 16)).reshape(
        NS * CH16, CHUNK, 16)

    x_p = jnp.concatenate([x, jnp.zeros((NPAD - N, D_IN), jnp.float32)])
    zeros_h1 = jnp.zeros((NPAD, H_IN), jnp.float32)
    zeros_h2 = jnp.zeros((NPAD, H_OUT), jnp.float32)

    degp = _deg_kernel(dst2, ew2)
    dis, xss = _prep_call(degp, x_p)

    acc1 = _agg1(xss, src3, dst3, wrep, zeros_h1)
    zss = _mid_call(acc1, xss, dis, W1, b1.reshape(1, D_H), W2)

    acc2 = _agg2(zss, src3, dst3, wrep, zeros_h2)
    out = _fin_call(acc2, zss, dis, b2.reshape(1, D_OUT))
    return out[:N]


# speedup vs baseline: 14.9927x; 1.6118x over previous
import functools

import jax
import jax.numpy as jnp
from jax import lax
from jax.experimental import pallas as pl
from jax.experimental.pallas import tpu as pltpu
from jax.experimental.pallas import tpu_sc as plsc

N = 10000
E = 320000
D_IN = 128
D_H = 256
D_OUT = 64

NPAD = 10240
NC = 2
NS = 16
NW = NC * NS
CHUNK = 128
CHUNKS = -(-E // (NW * CHUNK))
EPW = CHUNKS * CHUNK
EPAD = NW * EPW
CH16 = 2 * -(-E // (NS * CHUNK * 2))
ESUB = CH16 * CHUNK
EPAD16 = NS * ESUB
ROWS_PER_SUB = NPAD // NS

_mesh = plsc.VectorSubcoreMesh(
    core_axis_name="c", subcore_axis_name="s", num_cores=NC, num_subcores=NS)
_sc_params = pltpu.CompilerParams(
    needs_layout_passes=False, use_tc_tiling_on_sc=False)


@functools.partial(
    pl.kernel,
    out_type=jax.ShapeDtypeStruct((NW, NPAD), jnp.float32),
    mesh=_mesh,
    compiler_params=_sc_params,
    scratch_types=[
        pltpu.VMEM((EPW,), jnp.int32),
        pltpu.VMEM((EPW,), jnp.float32),
        pltpu.VMEM((NPAD,), jnp.float32),
    ],
)
def _deg_kernel(dst_hbm, ew_hbm, out_hbm, dstb, ewb, degl):
    cid = lax.axis_index("c")
    sid = lax.axis_index("s")
    wid = sid * NC + cid
    pltpu.sync_copy(dst_hbm.at[wid], dstb)
    pltpu.sync_copy(ew_hbm.at[wid], ewb)

    def zero(i, _):
        degl[pl.ds(i * 16, 16)] = jnp.zeros((16,), jnp.float32)
        return 0

    lax.fori_loop(0, NPAD // 16, zero, 0)

    def body(g, _):
        idx = dstb[pl.ds(g * 16, 16)]
        w = ewb[pl.ds(g * 16, 16)]
        plsc.addupdate_scatter(degl, [idx], w)
        return 0

    lax.fori_loop(0, EPW // 16, body, 0)
    pltpu.sync_copy(degl, out_hbm.at[wid])


def _make_agg_kernel(D2):
    @functools.partial(
        pl.kernel,
        out_type=jax.ShapeDtypeStruct((NC, NPAD, D2), jnp.float32),
        mesh=_mesh,
        compiler_params=_sc_params,
        scratch_types=[
            pltpu.VMEM((CH16, CHUNK), jnp.int32),
            pltpu.VMEM((CH16, CHUNK), jnp.int32),
            pltpu.VMEM((CHUNK, 16), jnp.float32),
            pltpu.VMEM((CHUNK, 16), jnp.float32),
            pltpu.VMEM((CHUNK, D2), jnp.float32),
            pltpu.VMEM((CHUNK, D2), jnp.float32),
            pltpu.VMEM_SHARED((NPAD, D2), jnp.float32),
            pltpu.SemaphoreType.DMA,
            pltpu.SemaphoreType.DMA,
            pltpu.SemaphoreType.DMA,
            pltpu.SemaphoreType.DMA,
        ],
    )
    def agg(table_hbm, src_hbm, dst_hbm, w_hbm, zeros_hbm, out_hbm,
            srcb, dstb, wbuf0, wbuf1, rows0, rows1, acc, g0, g1, w0, w1):
        cid = lax.axis_index("c")
        sid = lax.axis_index("s")
        rbase = sid * ROWS_PER_SUB
        pltpu.sync_copy(zeros_hbm.at[pl.ds(rbase, ROWS_PER_SUB)],
                        acc.at[pl.ds(rbase, ROWS_PER_SUB)])
        pltpu.sync_copy(src_hbm.at[sid], srcb)
        pltpu.sync_copy(dst_hbm.at[sid], dstb)
        plsc.subcore_barrier()

        def start(ci, rows, wbuf, gsem, wsem):
            pltpu.async_copy(table_hbm.at[cid].at[srcb.at[ci]], rows, gsem)
            pltpu.async_copy(w_hbm.at[sid * CH16 + ci], wbuf, wsem)

        def finish(ci, rows, wbuf, gsem, wsem):
            pltpu.make_async_copy(
                table_hbm.at[cid].at[srcb.at[ci]], rows, gsem).wait()
            pltpu.make_async_copy(
                w_hbm.at[sid * CH16 + ci], wbuf, wsem).wait()

            def scale(e, _):
                wv = wbuf[e, :]
                for j in range(D2 // 16):
                    sl = pl.ds(j * 16, 16)
                    rows[e, sl] = rows[e, sl] * wv
                return 0

            lax.fori_loop(0, CHUNK, scale, 0)
            pltpu.sync_copy(rows, acc.at[dstb.at[ci]], add=True)

        start(0, rows0, wbuf0, g0, w0)
        start(1, rows1, wbuf1, g1, w1)

        def body(g, _):
            a = 2 * g
            finish(a, rows0, wbuf0, g0, w0)
            start(a + 2, rows0, wbuf0, g0, w0)
            finish(a + 1, rows1, wbuf1, g1, w1)
            start(a + 3, rows1, wbuf1, g1, w1)
            return 0

        lax.fori_loop(0, CH16 // 2 - 1, body, 0)
        finish(CH16 - 2, rows0, wbuf0, g0, w0)
        finish(CH16 - 1, rows1, wbuf1, g1, w1)
        plsc.subcore_barrier()
        pltpu.sync_copy(acc.at[pl.ds(rbase, ROWS_PER_SUB)],
                        out_hbm.at[cid, pl.ds(rbase, ROWS_PER_SUB)])

    return agg


_agg1 = _make_agg_kernel(D_IN // 2)
_agg2 = _make_agg_kernel(D_OUT // 2)


RB = 1024
NBB = NPAD // RB


H_IN = D_IN // 2
H_OUT = D_OUT // 2


def _prep_body(degp_ref, x_ref, dis_ref, xs_ref):
    deg = jnp.sum(degp_ref[...], axis=0, keepdims=True) + 1.0
    disr = lax.rsqrt(deg)
    r = lax.broadcasted_iota(jnp.int32, (RB, RB), 0)
    c = lax.broadcasted_iota(jnp.int32, (RB, RB), 1)
    dis_col = jnp.sum(
        jnp.where(r == c, jnp.broadcast_to(disr, (RB, RB)), 0.0),
        axis=1, keepdims=True)
    dis_ref[...] = dis_col
    xs = x_ref[...] * dis_col
    xs_ref[0] = xs[:, :H_IN]
    xs_ref[1] = xs[:, H_IN:]


def _mid_body(acc_ref, xs_ref, dis_ref, w1_ref, b1_ref, w2_ref, zs_ref):
    dis = dis_ref[...]
    t = jnp.concatenate(
        [acc_ref[0] + xs_ref[0], acc_ref[1] + xs_ref[1]], axis=1) * dis
    h = jnp.maximum(
        jnp.dot(t, w1_ref[...], preferred_element_type=jnp.float32)
        + b1_ref[...], 0.0)
    z = jnp.dot(h, w2_ref[...], preferred_element_type=jnp.float32)
    zs = z * dis
    zs_ref[0] = zs[:, :H_OUT]
    zs_ref[1] = zs[:, H_OUT:]


def _fin_body(acc_ref, zs_ref, dis_ref, b2_ref, out_ref):
    t = jnp.concatenate(
        [acc_ref[0] + zs_ref[0], acc_ref[1] + zs_ref[1]], axis=1)
    out_ref[...] = t * dis_ref[...] + b2_ref[...]


_prep_call = pl.pallas_call(
    _prep_body,
    grid=(NBB,),
    in_specs=[
        pl.BlockSpec((NW, RB), lambda r: (0, r)),
        pl.BlockSpec((RB, D_IN), lambda r: (r, 0)),
    ],
    out_specs=[
        pl.BlockSpec((RB, 1), lambda r: (r, 0)),
        pl.BlockSpec((NC, RB, H_IN), lambda r: (0, r, 0)),
    ],
    out_shape=[
        jax.ShapeDtypeStruct((NPAD, 1), jnp.float32),
        jax.ShapeDtypeStruct((NC, NPAD, H_IN), jnp.float32),
    ],
)

_mid_call = pl.pallas_call(
    _mid_body,
    grid=(NBB,),
    in_specs=[
        pl.BlockSpec((NC, RB, H_IN), lambda r: (0, r, 0)),
        pl.BlockSpec((NC, RB, H_IN), lambda r: (0, r, 0)),
        pl.BlockSpec((RB, 1), lambda r: (r, 0)),
        pl.BlockSpec((D_IN, D_H), lambda r: (0, 0)),
        pl.BlockSpec((1, D_H), lambda r: (0, 0)),
        pl.BlockSpec((D_H, D_OUT), lambda r: (0, 0)),
    ],
    out_specs=pl.BlockSpec((NC, RB, H_OUT), lambda r: (0, r, 0)),
    out_shape=jax.ShapeDtypeStruct((NC, NPAD, H_OUT), jnp.float32),
)

_fin_call = pl.pallas_call(
    _fin_body,
    grid=(NBB,),
    in_specs=[
        pl.BlockSpec((NC, RB, H_OUT), lambda r: (0, r, 0)),
        pl.BlockSpec((NC, RB, H_OUT), lambda r: (0, r, 0)),
        pl.BlockSpec((RB, 1), lambda r: (r, 0)),
        pl.BlockSpec((1, D_OUT), lambda r: (0, 0)),
    ],
    out_specs=pl.BlockSpec((RB, D_OUT), lambda r: (r, 0)),
    out_shape=jax.ShapeDtypeStruct((NPAD, D_OUT), jnp.float32),
)


def kernel(x, edge_index, edge_weight, W1, b1, W2, b2):
    src = edge_index[0].astype(jnp.int32)
    dst = edge_index[1].astype(jnp.int32)
    ew = edge_weight.astype(jnp.float32)

    pad = EPAD - E
    dst2 = jnp.concatenate(
        [dst, jnp.zeros((pad,), jnp.int32)]).reshape(NW, EPW)
    ew2 = jnp.concatenate(
        [ew, jnp.zeros((pad,), jnp.float32)]).reshape(NW, EPW)

    pad16 = EPAD16 - E
    src3 = jnp.concatenate(
        [src, jnp.zeros((pad16,), jnp.int32)]).reshape(NS, CH16, CHUNK)
    dst3 = jnp.concatenate(
        [dst, jnp.zeros((pad16,), jnp.int32)]).reshape(NS, CH16, CHUNK)
    ew16 = jnp.concatenate([ew, jnp.zeros((pad16,), jnp.float32)])
    wrep = jnp.broadcast_to(ew16[:, None], (EPAD16, 16)).reshape(
        NS * CH16, CHUNK, 16)

    x_p = jnp.concatenate([x, jnp.zeros((NPAD - N, D_IN), jnp.float32)])
    zeros_h1 = jnp.zeros((NPAD, H_IN), jnp.float32)
    zeros_h2 = jnp.zeros((NPAD, H_OUT), jnp.float32)

    degp = _deg_kernel(dst2, ew2)
    dis, xss = _prep_call(degp, x_p)

    acc1 = _agg1(xss, src3, dst3, wrep, zeros_h1)
    zss = _mid_call(acc1, xss, dis, W1, b1.reshape(1, D_H), W2)

    acc2 = _agg2(zss, src3, dst3, wrep, zeros_h2)
    out = _fin_call(acc2, zss, dis, b2.reshape(1, D_OUT))
    return out[:N]
